# baseline (speedup 1.0000x reference)
import jax
import jax.numpy as jnp
from jax import lax
from jax.experimental import pallas as pl
from jax.experimental.pallas import tpu as pltpu
from jax.experimental.pallas import tpu_sc as plsc

B, T, D = 16, 512, 256
MAX_LEN = 2048
L = 16
CHUNK = 64
ROWS_PER_W = B * MAX_LEN // 32
NCHUNK = ROWS_PER_W // CHUNK
KG = CHUNK // L
CUNROLL = 16


def _lr_body(x_hbm, dur_hbm, out_hbm,
             dur_v, cum_v, idx_v, rank_v, cidx_v,
             cmp0, cmp1, rows0, rows1, zbuf,
             gsem0, gsem1, osem0, osem1):
    cid = lax.axis_index("c")
    sid = lax.axis_index("s")
    b = sid
    r0 = ((cid + sid) % 2) * ROWS_PER_W
    out_base = b * MAX_LEN + r0

    pltpu.sync_copy(dur_hbm.at[pl.ds(b * T, T)], dur_v)

    zero_v = jnp.zeros((L,), jnp.float32)

    def z_body(r, carry):
        for k in range(D // L):
            zbuf[r, pl.ds(k * L, L)] = zero_v
        return carry

    lax.fori_loop(0, CHUNK, z_body, 0)

    def cs_body(j, carry):
        v = jnp.maximum(dur_v[pl.ds(j * L, L)], 0)
        s = plsc.cumsum(v) + carry
        cum_v[pl.ds(j * L, L)] = s
        return s[L - 1]

    total = lax.fori_loop(0, T // L, cs_body, jnp.int32(0))
    limit = jnp.minimum(total, MAX_LEN)

    lane = lax.iota(jnp.int32, L)
    one = jnp.full((L,), 1, jnp.int32)
    zero = jnp.zeros((L,), jnp.int32)

    def issue_chunk(cd, cmpbuf, gsem):
        gpos0 = r0 + cd * CHUNK

        @pl.when(gpos0 < limit)
        def _():
            def jbody(j, carry):
                gofs = cd * CHUNK + j * L
                pos = gpos0 + j * L + lane
                base = zero
                for half in (256, 128, 64, 32, 16, 8, 4, 2, 1):
                    val = plsc.load_gather(cum_v, [base + (half - 1)])
                    base = base + jnp.where(val <= pos, half, 0)
                val = plsc.load_gather(cum_v, [base])
                cnt = base + jnp.where(val <= pos, 1, 0)
                gi = b * T + jnp.minimum(cnt, T - 1)
                idx_v[pl.ds(gofs, L)] = gi
                prev = cd * CHUNK + jnp.maximum(j * L + lane - 1, 0)
                pv = plsc.load_gather(idx_v, [prev])
                rk = plsc.cumsum(jnp.where(gi != pv, one, zero)) + carry
                rank_v[pl.ds(gofs, L)] = rk
                plsc.store_scatter(cidx_v, [cd * CHUNK + rk], gi)
                return rk[L - 1]

            u1 = lax.fori_loop(0, KG, jbody, jnp.int32(0))
            for j in range(KG):
                src = cd * CHUNK + lax.rem(j * L + lane, u1 + 1)
                cidx_v[pl.ds(cd * CHUNK + j * L, L)] = (
                    plsc.load_gather(cidx_v, [src]))
            u = u1 + 1
            for k in range(KG):
                @pl.when(u > k * L)
                def _g(k=k):
                    pltpu.async_copy(
                        x_hbm.at[cidx_v.at[pl.ds(cd * CHUNK + k * L, L)]],
                        cmpbuf.at[pl.ds(k * L, L)], gsem)

    def finish_chunk(cd, cmpbuf, rowsbuf, gsem, osem):
        gpos0 = r0 + cd * CHUNK
        live = gpos0 < limit
        dst = out_hbm.at[pl.ds(out_base + cd * CHUNK, CHUNK)]

        @pl.when(live)
        def _():
            u1 = rank_v[pl.ds(cd * CHUNK + CHUNK - L, L)][L - 1]
            u = u1 + 1
            for k in range(KG):
                @pl.when(u > k * L)
                def _w(k=k):
                    pltpu.make_async_copy(
                        x_hbm.at[cidx_v.at[pl.ds(cd * CHUNK + k * L, L)]],
                        cmpbuf.at[pl.ds(k * L, L)], gsem).wait()

            def ebody(j, carry):
                rks = rank_v[pl.ds(cd * CHUNK + j * L, L)]
                drows = j * L + lane

                def cgroup(cg, carry2):
                    for cc in range(CUNROLL):
                        col = cg * CUNROLL + cc
                        colv = zero + col
                        v = plsc.load_gather(cmpbuf, [rks, colv])
                        plsc.store_scatter(rowsbuf, [drows, colv], v)
                    return carry2

                lax.fori_loop(0, D // CUNROLL, cgroup, 0)
                return carry

            lax.fori_loop(0, KG, ebody, 0)

            mstart = jnp.clip(limit - gpos0, 0, CHUNK)

            def zrow(r, carry):
                for k in range(D // L):
                    rowsbuf[r, pl.ds(k * L, L)] = zero_v
                return carry

            lax.fori_loop(mstart, CHUNK, zrow, 0)
            pltpu.async_copy(rowsbuf, dst, osem)

        @pl.when(jnp.logical_not(live))
        def _masked():
            pltpu.async_copy(zbuf, dst, osem)

    def drain_out(cd, rowsbuf, osem):
        pltpu.make_async_copy(
            rowsbuf, out_hbm.at[pl.ds(out_base + cd * CHUNK, CHUNK)],
            osem).wait()

    issue_chunk(0, cmp0, gsem0)
    issue_chunk(1, cmp1, gsem1)
    finish_chunk(0, cmp0, rows0, gsem0, osem0)
    issue_chunk(2, cmp0, gsem0)
    finish_chunk(1, cmp1, rows1, gsem1, osem1)
    issue_chunk(3, cmp1, gsem1)
    drain_out(0, rows0, osem0)
    finish_chunk(2, cmp0, rows0, gsem0, osem0)

    def gbody(g, carry):
        codd = 3 + 2 * g
        issue_chunk(codd + 1, cmp0, gsem0)
        drain_out(codd - 2, rows1, osem1)
        finish_chunk(codd, cmp1, rows1, gsem1, osem1)
        ceven = codd + 1
        issue_chunk(ceven + 1, cmp1, gsem1)
        drain_out(ceven - 2, rows0, osem0)
        finish_chunk(ceven, cmp0, rows0, gsem0, osem0)
        return carry

    lax.fori_loop(0, (NCHUNK - 4) // 2, gbody, 0)

    drain_out(NCHUNK - 3, rows1, osem1)
    finish_chunk(NCHUNK - 1, cmp1, rows1, gsem1, osem1)
    drain_out(NCHUNK - 2, rows0, osem0)
    drain_out(NCHUNK - 1, rows1, osem1)


def kernel(x, durations, max_len):
    xflat = x.reshape(B * T, D)
    durflat = durations.reshape(B * T).astype(jnp.int32)
    mesh = plsc.VectorSubcoreMesh(core_axis_name="c", subcore_axis_name="s",
                                  num_cores=2, num_subcores=16)
    run = pl.kernel(
        _lr_body,
        out_type=jax.ShapeDtypeStruct((B * MAX_LEN, D), jnp.float32),
        mesh=mesh,
        scratch_types=[
            pltpu.VMEM((T,), jnp.int32),
            pltpu.VMEM((T,), jnp.int32),
            pltpu.VMEM((ROWS_PER_W,), jnp.int32),
            pltpu.VMEM((ROWS_PER_W,), jnp.int32),
            pltpu.VMEM((ROWS_PER_W,), jnp.int32),
            pltpu.VMEM((CHUNK, D), jnp.float32),
            pltpu.VMEM((CHUNK, D), jnp.float32),
            pltpu.VMEM((CHUNK, D), jnp.float32),
            pltpu.VMEM((CHUNK, D), jnp.float32),
            pltpu.VMEM((CHUNK, D), jnp.float32),
            pltpu.SemaphoreType.DMA,
            pltpu.SemaphoreType.DMA,
            pltpu.SemaphoreType.DMA,
            pltpu.SemaphoreType.DMA,
        ],
        compiler_params=pltpu.CompilerParams(needs_layout_passes=False),
    )
    out = run(xflat, durflat)
    return out.reshape(B, MAX_LEN, D)


# speedup vs baseline: 3.1664x; 3.1664x over previous
import jax
import jax.numpy as jnp
from jax import lax
from jax.experimental import pallas as pl
from jax.experimental.pallas import tpu as pltpu
from jax.experimental.pallas import tpu_sc as plsc

B, T, D = 16, 512, 256
MAX_LEN = 2048
L = 16
CHUNK = 64
ROWS_PER_W = B * MAX_LEN // 32
NCHUNK = ROWS_PER_W // CHUNK
KG = CHUNK // L
CUNROLL = 16


def _lr_body(x_hbm, dur_hbm, out_hbm,
             dur_v, cum_v, idx_v, rank_v, cidx_v,
             cmp0, cmp1, rows0, rows1, zbuf,
             gsem0, gsem1, osem0, osem1):
    cid = lax.axis_index("c")
    sid = lax.axis_index("s")
    b = sid
    r0 = ((cid + sid) % 2) * ROWS_PER_W
    out_base = b * MAX_LEN + r0

    pltpu.sync_copy(dur_hbm.at[pl.ds(b * T, T)], dur_v)

    zero_v = jnp.zeros((L,), jnp.float32)

    def z_body(r, carry):
        for k in range(D // L):
            zbuf[r, pl.ds(k * L, L)] = zero_v
        return carry

    lax.fori_loop(0, CHUNK, z_body, 0)

    def cs_body(j, carry):
        v = jnp.maximum(dur_v[pl.ds(j * L, L)], 0)
        s = plsc.cumsum(v) + carry
        cum_v[pl.ds(j * L, L)] = s
        return s[L - 1]

    total = lax.fori_loop(0, T // L, cs_body, jnp.int32(0))
    limit = jnp.minimum(total, MAX_LEN)

    lane = lax.iota(jnp.int32, L)
    one = jnp.full((L,), 1, jnp.int32)
    zero = jnp.zeros((L,), jnp.int32)

    def issue_chunk(cd, cmpbuf, gsem):
        gpos0 = r0 + cd * CHUNK

        @pl.when(gpos0 < limit)
        def _():
            def jbody(j, carry):
                gofs = cd * CHUNK + j * L
                pos = gpos0 + j * L + lane
                base = zero
                for half in (256, 128, 64, 32, 16, 8, 4, 2, 1):
                    val = plsc.load_gather(cum_v, [base + (half - 1)])
                    base = base + jnp.where(val <= pos, half, 0)
                val = plsc.load_gather(cum_v, [base])
                cnt = base + jnp.where(val <= pos, 1, 0)
                gi = b * T + jnp.minimum(cnt, T - 1)
                idx_v[pl.ds(gofs, L)] = gi
                prev = cd * CHUNK + jnp.maximum(j * L + lane - 1, 0)
                pv = plsc.load_gather(idx_v, [prev])
                rk = plsc.cumsum(jnp.where(gi != pv, one, zero)) + carry
                rank_v[pl.ds(gofs, L)] = rk
                plsc.store_scatter(cidx_v, [cd * CHUNK + rk], gi)
                return rk[L - 1]

            u1 = lax.fori_loop(0, KG, jbody, jnp.int32(0))
            for j in range(KG):
                src = cd * CHUNK + lax.rem(j * L + lane, u1 + 1)
                cidx_v[pl.ds(cd * CHUNK + j * L, L)] = (
                    plsc.load_gather(cidx_v, [src]))
            u = u1 + 1
            for k in range(KG):
                @pl.when(u > k * L)
                def _g(k=k):
                    pltpu.async_copy(
                        x_hbm.at[cidx_v.at[pl.ds(cd * CHUNK + k * L, L)]],
                        cmpbuf.at[pl.ds(k * L, L)], gsem)

    def finish_chunk(cd, cmpbuf, rowsbuf, gsem, osem):
        gpos0 = r0 + cd * CHUNK
        live = gpos0 < limit
        dst = out_hbm.at[pl.ds(out_base + cd * CHUNK, CHUNK)]

        @pl.when(live)
        def _():
            u1 = rank_v[pl.ds(cd * CHUNK + CHUNK - L, L)][L - 1]
            u = u1 + 1
            for k in range(KG):
                @pl.when(u > k * L)
                def _w(k=k):
                    pltpu.make_async_copy(
                        x_hbm.at[cidx_v.at[pl.ds(cd * CHUNK + k * L, L)]],
                        cmpbuf.at[pl.ds(k * L, L)], gsem).wait()

            def ebody(j, carry):
                rks = rank_v[pl.ds(cd * CHUNK + j * L, L)]
                drows = j * L + lane

                def cgroup(cg, carry2):
                    for t in range(L):
                        colv = cg * L + jnp.bitwise_and(lane + t, L - 1)
                        v = plsc.load_gather(cmpbuf, [rks, colv])
                        plsc.store_scatter(rowsbuf, [drows, colv], v)
                    return carry2

                lax.fori_loop(0, D // L, cgroup, 0)
                return carry

            lax.fori_loop(0, KG, ebody, 0)

            mstart = jnp.clip(limit - gpos0, 0, CHUNK)

            def zrow(r, carry):
                for k in range(D // L):
                    rowsbuf[r, pl.ds(k * L, L)] = zero_v
                return carry

            lax.fori_loop(mstart, CHUNK, zrow, 0)
            pltpu.async_copy(rowsbuf, dst, osem)

        @pl.when(jnp.logical_not(live))
        def _masked():
            pltpu.async_copy(zbuf, dst, osem)

    def drain_out(cd, rowsbuf, osem):
        pltpu.make_async_copy(
            rowsbuf, out_hbm.at[pl.ds(out_base + cd * CHUNK, CHUNK)],
            osem).wait()

    issue_chunk(0, cmp0, gsem0)
    issue_chunk(1, cmp1, gsem1)
    finish_chunk(0, cmp0, rows0, gsem0, osem0)
    issue_chunk(2, cmp0, gsem0)
    finish_chunk(1, cmp1, rows1, gsem1, osem1)
    issue_chunk(3, cmp1, gsem1)
    drain_out(0, rows0, osem0)
    finish_chunk(2, cmp0, rows0, gsem0, osem0)

    def gbody(g, carry):
        codd = 3 + 2 * g
        issue_chunk(codd + 1, cmp0, gsem0)
        drain_out(codd - 2, rows1, osem1)
        finish_chunk(codd, cmp1, rows1, gsem1, osem1)
        ceven = codd + 1
        issue_chunk(ceven + 1, cmp1, gsem1)
        drain_out(ceven - 2, rows0, osem0)
        finish_chunk(ceven, cmp0, rows0, gsem0, osem0)
        return carry

    lax.fori_loop(0, (NCHUNK - 4) // 2, gbody, 0)

    drain_out(NCHUNK - 3, rows1, osem1)
    finish_chunk(NCHUNK - 1, cmp1, rows1, gsem1, osem1)
    drain_out(NCHUNK - 2, rows0, osem0)
    drain_out(NCHUNK - 1, rows1, osem1)


def kernel(x, durations, max_len):
    xflat = x.reshape(B * T, D)
    durflat = durations.reshape(B * T).astype(jnp.int32)
    mesh = plsc.VectorSubcoreMesh(core_axis_name="c", subcore_axis_name="s",
                                  num_cores=2, num_subcores=16)
    run = pl.kernel(
        _lr_body,
        out_type=jax.ShapeDtypeStruct((B * MAX_LEN, D), jnp.float32),
        mesh=mesh,
        scratch_types=[
            pltpu.VMEM((T,), jnp.int32),
            pltpu.VMEM((T,), jnp.int32),
            pltpu.VMEM((ROWS_PER_W,), jnp.int32),
            pltpu.VMEM((ROWS_PER_W,), jnp.int32),
            pltpu.VMEM((ROWS_PER_W,), jnp.int32),
            pltpu.VMEM((CHUNK, D), jnp.float32),
            pltpu.VMEM((CHUNK, D), jnp.float32),
            pltpu.VMEM((CHUNK, D), jnp.float32),
            pltpu.VMEM((CHUNK, D), jnp.float32),
            pltpu.VMEM((CHUNK, D), jnp.float32),
            pltpu.SemaphoreType.DMA,
            pltpu.SemaphoreType.DMA,
            pltpu.SemaphoreType.DMA,
            pltpu.SemaphoreType.DMA,
        ],
        compiler_params=pltpu.CompilerParams(needs_layout_passes=False),
    )
    out = run(xflat, durflat)
    return out.reshape(B, MAX_LEN, D)


# speedup vs baseline: 6.5222x; 2.0598x over previous
import jax
import jax.numpy as jnp
from jax import lax
from jax.experimental import pallas as pl
from jax.experimental.pallas import tpu as pltpu
from jax.experimental.pallas import tpu_sc as plsc

B, T, D = 16, 512, 256
MAX_LEN = 2048
L = 16
CHUNK = 128
ROWS_PER_W = B * MAX_LEN // 32
NCHUNK = ROWS_PER_W // CHUNK
NBUF = 3
ZROWS = 64


def _lr_body(x_hbm, dur_hbm, out_hbm,
             dur_v, cum_v, idx_v,
             rows_v0, rows_v1, rows_v2, zbuf,
             gsem0, gsem1, gsem2, osem0, osem1, osem2):
    cid = lax.axis_index("c")
    sid = lax.axis_index("s")
    b = sid
    r0 = ((cid + sid) % 2) * ROWS_PER_W

    bufs = (rows_v0, rows_v1, rows_v2)
    gsems = (gsem0, gsem1, gsem2)
    osems = (osem0, osem1, osem2)

    pltpu.sync_copy(dur_hbm.at[b], dur_v)

    zero_v = jnp.zeros((L,), jnp.float32)

    def cs_body(j, carry):
        v = jnp.maximum(dur_v[pl.ds(j * L, L)], 0)
        s = plsc.cumsum(v) + carry
        cum_v[pl.ds(j * L, L)] = s
        return s[L - 1]

    total = lax.fori_loop(0, T // L, cs_body, jnp.int32(0))
    limit = jnp.minimum(total, MAX_LEN)

    @pl.when(limit < r0 + ROWS_PER_W)
    def _zinit():
        def z_body(r, carry):
            for k in range(D // L):
                zbuf[r, pl.ds(k * L, L)] = zero_v
            return carry

        lax.fori_loop(0, ZROWS, z_body, 0)

    lane = lax.iota(jnp.int32, L)

    def compute_idx(c):
        def ss_body(j, _):
            pos = r0 + c * CHUNK + j * L + lane
            base = jnp.zeros((L,), jnp.int32)
            for half in (256, 128, 64, 32, 16, 8, 4, 2, 1):
                val = plsc.load_gather(cum_v, [base + (half - 1)])
                base = base + jnp.where(val <= pos, half, 0)
            val = plsc.load_gather(cum_v, [base])
            cnt = base + jnp.where(val <= pos, 1, 0)
            src = jnp.where(cnt > T - 1, jnp.bitwise_and(pos, T - 1), cnt)
            idx_v[pl.ds(c * CHUNK + j * L, L)] = b * T + src
            return 0

        lax.fori_loop(0, CHUNK // L, ss_body, 0)

    out_base = b * MAX_LEN + r0

    def finish_chunk(c):
        buf = bufs[c % NBUF]
        gpos0 = r0 + c * CHUNK
        live = gpos0 < limit
        dst = out_hbm.at[pl.ds(out_base + c * CHUNK, CHUNK)]

        @pl.when(live)
        def _live():
            pltpu.make_async_copy(
                x_hbm.at[idx_v.at[pl.ds(c * CHUNK, CHUNK)]],
                buf, gsems[c % NBUF]).wait()
            mstart = jnp.clip(limit - gpos0, 0, CHUNK)

            def zrow(r, carry):
                for k in range(D // L):
                    buf[r, pl.ds(k * L, L)] = zero_v
                return carry

            lax.fori_loop(mstart, CHUNK, zrow, 0)
            pltpu.async_copy(buf, dst, osems[c % NBUF])

        @pl.when(jnp.logical_not(live))
        def _masked():
            pltpu.async_copy(zbuf, dst.at[pl.ds(0, ZROWS)], osems[c % NBUF])
            pltpu.async_copy(zbuf, dst.at[pl.ds(ZROWS, ZROWS)],
                             osems[c % NBUF])

    def drain_out(c):
        pltpu.make_async_copy(
            bufs[c % NBUF],
            out_hbm.at[pl.ds(out_base + c * CHUNK, CHUNK)],
            osems[c % NBUF]).wait()

    def issue_chunk(c):
        if c >= NBUF:
            drain_out(c - NBUF)

        @pl.when(r0 + c * CHUNK < limit)
        def _issue():
            compute_idx(c)
            pltpu.async_copy(
                x_hbm.at[idx_v.at[pl.ds(c * CHUNK, CHUNK)]],
                bufs[c % NBUF], gsems[c % NBUF])

    issue_chunk(0)
    issue_chunk(1)
    for c in range(NCHUNK):
        if c + 2 < NCHUNK:
            issue_chunk(c + 2)
        finish_chunk(c)
    for c in range(NCHUNK - NBUF, NCHUNK):
        drain_out(c)


def kernel(x, durations, max_len):
    xflat = x.reshape(B * T, D)
    durflat = durations
    mesh = plsc.VectorSubcoreMesh(core_axis_name="c", subcore_axis_name="s",
                                  num_cores=2, num_subcores=16)
    run = pl.kernel(
        _lr_body,
        out_type=jax.ShapeDtypeStruct((B * MAX_LEN, D), jnp.float32),
        mesh=mesh,
        scratch_types=[
            pltpu.VMEM((T,), jnp.int32),
            pltpu.VMEM((T,), jnp.int32),
            pltpu.VMEM((ROWS_PER_W,), jnp.int32),
            pltpu.VMEM((CHUNK, D), jnp.float32),
            pltpu.VMEM((CHUNK, D), jnp.float32),
            pltpu.VMEM((CHUNK, D), jnp.float32),
            pltpu.VMEM((ZROWS, D), jnp.float32),
            pltpu.SemaphoreType.DMA,
            pltpu.SemaphoreType.DMA,
            pltpu.SemaphoreType.DMA,
            pltpu.SemaphoreType.DMA,
            pltpu.SemaphoreType.DMA,
            pltpu.SemaphoreType.DMA,
        ],
        compiler_params=pltpu.CompilerParams(needs_layout_passes=False),
    )
    out = run(xflat, durflat)
    return out.reshape(B, MAX_LEN, D)
